# baseline (speedup 1.0000x reference)
import functools

import jax
import jax.numpy as jnp
from jax import lax
from jax.experimental import pallas as pl
from jax.experimental.pallas import tpu as pltpu
from jax.experimental.pallas import tpu_sc as plsc

L_E = 32
N_ROWS = 729
D = 1152
EPS = 1e-8

_RC = 81
_LC = 128
_NR = N_ROWS // _RC
_NL = D // _LC

_SC_TILES = 27
_SC_ROWS = N_ROWS // _SC_TILES
_SC_RG = 9
_SC_CHUNKS = (_SC_ROWS // _SC_RG) * _NL


def _stats_body(h_ref, x_ref, s_ref, dacc, qacc):
    j = pl.program_id(0)
    r = pl.program_id(1)
    x = x_ref[...]
    h = h_ref[...]

    @pl.when(jnp.logical_and(j == 0, r == 0))
    def _():
        dacc[...] = jnp.zeros_like(dacc)
        qacc[...] = jnp.zeros_like(qacc)

    dacc[...] += jnp.sum(x * h, axis=0)
    qacc[...] += jnp.sum(x * x, axis=0)

    @pl.when(jnp.logical_and(j == _NL - 1, r == _NR - 1))
    def _():
        dots = jnp.sum(dacc[...], axis=1)
        sqs = jnp.sum(qacc[...], axis=1)
        s_ref[...] = dots / (jnp.sqrt(sqs) + EPS)


def _tc_stats(mem_t, h3):
    return pl.pallas_call(
        _stats_body,
        grid=(_NL, _NR),
        in_specs=[
            pl.BlockSpec((_RC, 1, _LC), lambda j, r: (r, 0, j)),
            pl.BlockSpec((_RC, L_E, _LC), lambda j, r: (r, 0, j)),
        ],
        out_specs=pl.BlockSpec((L_E,), lambda j, r: (0,)),
        out_shape=jax.ShapeDtypeStruct((L_E,), jnp.float32),
        scratch_shapes=[
            pltpu.VMEM((L_E, _LC), jnp.float32),
            pltpu.VMEM((L_E, _LC), jnp.float32),
        ],
    )(h3, mem_t)


_SC_MESH = plsc.VectorSubcoreMesh(core_axis_name="c", subcore_axis_name="s")


@functools.partial(
    pl.kernel,
    mesh=_SC_MESH,
    out_type=jax.ShapeDtypeStruct((N_ROWS, L_E, D), jnp.float32),
    scratch_types=[
        pltpu.VMEM((_SC_RG, L_E, _LC), jnp.float32),
        pltpu.VMEM((_SC_RG, L_E, _LC), jnp.float32),
        pltpu.SemaphoreType.DMA,
        pltpu.SemaphoreType.DMA,
        pltpu.SemaphoreType.DMA,
        pltpu.SemaphoreType.DMA,
    ],
)
def _sc_copy(mem_hbm, out_hbm, buf0, buf1, rs0, rs1, ws0, ws1):
    c = lax.axis_index("c")
    s = lax.axis_index("s")
    wid = s * 2 + c

    @pl.when(wid < _SC_TILES)
    def _():
        bufs = (buf0, buf1)
        rsems = (rs0, rs1)
        wsems = (ws0, ws1)
        writes = [None, None]
        for k in range(_SC_CHUNKS):
            b = k % 2
            rr = k % (_SC_ROWS // _SC_RG)
            j = k // (_SC_ROWS // _SC_RG)
            r0 = wid * _SC_ROWS + rr * _SC_RG
            c0 = j * _LC
            if writes[b] is not None:
                writes[b].wait()
            pltpu.async_copy(
                mem_hbm.at[pl.ds(r0, _SC_RG), :, pl.ds(c0, _LC)],
                bufs[b],
                rsems[b],
            ).wait()
            writes[b] = pltpu.async_copy(
                bufs[b],
                out_hbm.at[pl.ds(r0, _SC_RG), :, pl.ds(c0, _LC)],
                wsems[b],
            )
        writes[0].wait()
        writes[1].wait()


@functools.partial(
    pl.kernel,
    mesh=_SC_MESH,
    out_type=jax.ShapeDtypeStruct((16,), jnp.int32),
    scratch_types=[
        pltpu.VMEM((L_E,), jnp.float32),
        pltpu.VMEM((16,), jnp.int32),
    ],
)
def _sc_argmax(scores_hbm, idx_hbm, scores_v, idx_v):
    c = lax.axis_index("c")
    s = lax.axis_index("s")
    wid = s * 2 + c

    @pl.when(wid == 0)
    def _():
        pltpu.sync_copy(scores_hbm, scores_v)
        va = scores_v[pl.ds(0, 16)]
        vb = scores_v[pl.ds(16, 16)]
        svals = [va[j] for j in range(16)] + [vb[j] for j in range(16)]
        best = svals[0]
        idx = jnp.int32(0)
        for j in range(1, L_E):
            take = svals[j] > best
            best = jnp.where(take, svals[j], best)
            idx = jnp.where(take, jnp.int32(j), idx)
        idx_v[...] = jnp.broadcast_to(idx, (16,))
        pltpu.sync_copy(idx_v, idx_hbm)


def _scatter_body(idx_ref, oin_ref, h_ref, o_ref, buf, sem):
    del oin_ref
    idx = idx_ref[0]
    cp = pltpu.make_async_copy(h_ref, buf, sem)
    cp.start()
    cp.wait()
    cp2 = pltpu.make_async_copy(buf, o_ref.at[:, idx, :], sem)
    cp2.start()
    cp2.wait()


def _scatter(idx_arr, out_t, H_t):
    return pl.pallas_call(
        _scatter_body,
        in_specs=[
            pl.BlockSpec(memory_space=pltpu.SMEM),
            pl.BlockSpec(memory_space=pl.ANY),
            pl.BlockSpec(memory_space=pl.ANY),
        ],
        out_specs=pl.BlockSpec(memory_space=pl.ANY),
        out_shape=jax.ShapeDtypeStruct((N_ROWS, L_E, D), jnp.float32),
        scratch_shapes=[
            pltpu.VMEM((N_ROWS, D), jnp.float32),
            pltpu.SemaphoreType.DMA,
        ],
        input_output_aliases={1: 0},
    )(idx_arr, out_t, H_t)


def kernel(mem, H_t):
    mem_t = jnp.transpose(mem, (1, 0, 2))
    h3 = H_t[:, None, :]
    scores = _tc_stats(mem_t, h3)
    out_t = _sc_copy(mem_t)
    idx_arr = _sc_argmax(scores)
    out_t = _scatter(idx_arr, out_t, H_t)
    return jnp.transpose(out_t, (1, 0, 2))


# speedup vs baseline: 1.8179x; 1.8179x over previous
import functools

import jax
import jax.numpy as jnp
from jax import lax
from jax.experimental import pallas as pl
from jax.experimental.pallas import tpu as pltpu
from jax.experimental.pallas import tpu_sc as plsc

L_E = 32
N_ROWS = 729
D = 1152
EPS = 1e-8

_RC = 243
_LC = 128
_NR = N_ROWS // _RC
_NL = D // _LC


def _copy_stats_body(h_ref, x_ref, o_ref, s_ref, dacc, qacc):
    j = pl.program_id(0)
    r = pl.program_id(1)
    x = x_ref[...]
    o_ref[...] = x
    h = h_ref[...]

    @pl.when(jnp.logical_and(j == 0, r == 0))
    def _():
        dacc[...] = jnp.zeros_like(dacc)
        qacc[...] = jnp.zeros_like(qacc)

    dacc[...] += jnp.sum(x * h, axis=0)
    qacc[...] += jnp.sum(x * x, axis=0)

    @pl.when(jnp.logical_and(j == _NL - 1, r == _NR - 1))
    def _():
        dots = jnp.sum(dacc[...], axis=1)
        sqs = jnp.sum(qacc[...], axis=1)
        s_ref[...] = dots / (jnp.sqrt(sqs) + EPS)


def _pass_a(mem_t, h3):
    return pl.pallas_call(
        _copy_stats_body,
        grid=(_NL, _NR),
        in_specs=[
            pl.BlockSpec((_RC, 1, _LC), lambda j, r: (r, 0, j)),
            pl.BlockSpec((_RC, L_E, _LC), lambda j, r: (r, 0, j)),
        ],
        out_specs=[
            pl.BlockSpec((_RC, L_E, _LC), lambda j, r: (r, 0, j)),
            pl.BlockSpec((L_E,), lambda j, r: (0,)),
        ],
        out_shape=[
            jax.ShapeDtypeStruct((N_ROWS, L_E, D), jnp.float32),
            jax.ShapeDtypeStruct((L_E,), jnp.float32),
        ],
        scratch_shapes=[
            pltpu.VMEM((L_E, _LC), jnp.float32),
            pltpu.VMEM((L_E, _LC), jnp.float32),
        ],
    )(h3, mem_t)


_SC_MESH = plsc.VectorSubcoreMesh(core_axis_name="c", subcore_axis_name="s")


@functools.partial(
    pl.kernel,
    mesh=_SC_MESH,
    out_type=jax.ShapeDtypeStruct((16,), jnp.int32),
    scratch_types=[
        pltpu.VMEM((L_E,), jnp.float32),
        pltpu.VMEM((16,), jnp.int32),
    ],
)
def _sc_argmax(scores_hbm, idx_hbm, scores_v, idx_v):
    c = lax.axis_index("c")
    s = lax.axis_index("s")
    wid = s * 2 + c

    @pl.when(wid == 0)
    def _():
        pltpu.sync_copy(scores_hbm, scores_v)
        va = scores_v[pl.ds(0, 16)]
        vb = scores_v[pl.ds(16, 16)]
        svals = [va[j] for j in range(16)] + [vb[j] for j in range(16)]
        best = svals[0]
        idx = jnp.int32(0)
        for j in range(1, L_E):
            take = svals[j] > best
            best = jnp.where(take, svals[j], best)
            idx = jnp.where(take, jnp.int32(j), idx)
        idx_v[...] = jnp.broadcast_to(idx, (16,))
        pltpu.sync_copy(idx_v, idx_hbm)


def _scatter_body(idx_ref, oin_ref, h_ref, o_ref, buf, sem):
    del oin_ref
    idx = idx_ref[0]
    cp = pltpu.make_async_copy(h_ref, buf, sem)
    cp.start()
    cp.wait()
    cp2 = pltpu.make_async_copy(buf, o_ref.at[:, idx, :], sem)
    cp2.start()
    cp2.wait()


def _pass_c(idx_arr, out_t, H_t):
    return pl.pallas_call(
        _scatter_body,
        in_specs=[
            pl.BlockSpec(memory_space=pltpu.SMEM),
            pl.BlockSpec(memory_space=pl.ANY),
            pl.BlockSpec(memory_space=pl.ANY),
        ],
        out_specs=pl.BlockSpec(memory_space=pl.ANY),
        out_shape=jax.ShapeDtypeStruct((N_ROWS, L_E, D), jnp.float32),
        scratch_shapes=[
            pltpu.VMEM((N_ROWS, D), jnp.float32),
            pltpu.SemaphoreType.DMA,
        ],
        input_output_aliases={1: 0},
    )(idx_arr, out_t, H_t)


def kernel(mem, H_t):
    mem_t = jnp.transpose(mem, (1, 0, 2))
    h3 = H_t[:, None, :]
    out_t, scores = _pass_a(mem_t, h3)
    idx_arr = _sc_argmax(scores)
    out_t = _pass_c(idx_arr, out_t, H_t)
    return jnp.transpose(out_t, (1, 0, 2))


# speedup vs baseline: 1.8599x; 1.0231x over previous
import functools

import jax
import jax.numpy as jnp
from jax import lax
from jax.experimental import pallas as pl
from jax.experimental.pallas import tpu as pltpu
from jax.experimental.pallas import tpu_sc as plsc

L_E = 32
N_ROWS = 729
D = 1152
EPS = 1e-8

_RC = 27
_NR = N_ROWS // _RC


def _copy_stats_body(h_ref, x_ref, o_ref, s_ref, dacc, qacc):
    r = pl.program_id(0)
    x = x_ref[...]
    o_ref[...] = x
    h = h_ref[...]

    @pl.when(r == 0)
    def _():
        dacc[...] = jnp.zeros_like(dacc)
        qacc[...] = jnp.zeros_like(qacc)

    dacc[...] += jnp.sum(x * h, axis=0)
    qacc[...] += jnp.sum(x * x, axis=0)

    @pl.when(r == _NR - 1)
    def _():
        dots = jnp.sum(dacc[...], axis=1)
        sqs = jnp.sum(qacc[...], axis=1)
        s_ref[...] = dots / (jnp.sqrt(sqs) + EPS)


def _pass_a(mem_t, h3):
    return pl.pallas_call(
        _copy_stats_body,
        grid=(_NR,),
        in_specs=[
            pl.BlockSpec((_RC, 1, D), lambda r: (r, 0, 0)),
            pl.BlockSpec((_RC, L_E, D), lambda r: (r, 0, 0)),
        ],
        out_specs=[
            pl.BlockSpec((_RC, L_E, D), lambda r: (r, 0, 0)),
            pl.BlockSpec((L_E,), lambda r: (0,)),
        ],
        out_shape=[
            jax.ShapeDtypeStruct((N_ROWS, L_E, D), jnp.float32),
            jax.ShapeDtypeStruct((L_E,), jnp.float32),
        ],
        scratch_shapes=[
            pltpu.VMEM((L_E, D), jnp.float32),
            pltpu.VMEM((L_E, D), jnp.float32),
        ],
    )(h3, mem_t)


_SC_MESH = plsc.VectorSubcoreMesh(core_axis_name="c", subcore_axis_name="s")


@functools.partial(
    pl.kernel,
    mesh=_SC_MESH,
    out_type=jax.ShapeDtypeStruct((16,), jnp.int32),
    scratch_types=[
        pltpu.VMEM((L_E,), jnp.float32),
        pltpu.VMEM((16,), jnp.int32),
    ],
)
def _sc_argmax(scores_hbm, idx_hbm, scores_v, idx_v):
    c = lax.axis_index("c")
    s = lax.axis_index("s")
    wid = s * 2 + c

    @pl.when(wid == 0)
    def _():
        pltpu.sync_copy(scores_hbm, scores_v)
        va = scores_v[pl.ds(0, 16)]
        vb = scores_v[pl.ds(16, 16)]
        svals = [va[j] for j in range(16)] + [vb[j] for j in range(16)]
        best = svals[0]
        idx = jnp.int32(0)
        for j in range(1, L_E):
            take = svals[j] > best
            best = jnp.where(take, svals[j], best)
            idx = jnp.where(take, jnp.int32(j), idx)
        idx_v[...] = jnp.broadcast_to(idx, (16,))
        pltpu.sync_copy(idx_v, idx_hbm)


def _scatter_body(idx_ref, oin_ref, h_ref, o_ref, buf, sem):
    del oin_ref
    idx = idx_ref[0]
    cp = pltpu.make_async_copy(h_ref, buf, sem)
    cp.start()
    cp.wait()
    cp2 = pltpu.make_async_copy(buf, o_ref.at[:, idx, :], sem)
    cp2.start()
    cp2.wait()


def _pass_c(idx_arr, out_t, H_t):
    return pl.pallas_call(
        _scatter_body,
        in_specs=[
            pl.BlockSpec(memory_space=pltpu.SMEM),
            pl.BlockSpec(memory_space=pl.ANY),
            pl.BlockSpec(memory_space=pl.ANY),
        ],
        out_specs=pl.BlockSpec(memory_space=pl.ANY),
        out_shape=jax.ShapeDtypeStruct((N_ROWS, L_E, D), jnp.float32),
        scratch_shapes=[
            pltpu.VMEM((N_ROWS, D), jnp.float32),
            pltpu.SemaphoreType.DMA,
        ],
        input_output_aliases={1: 0},
    )(idx_arr, out_t, H_t)


def kernel(mem, H_t):
    mem_t = jnp.transpose(mem, (1, 0, 2))
    h3 = H_t[:, None, :]
    out_t, scores = _pass_a(mem_t, h3)
    idx_arr = _sc_argmax(scores)
    out_t = _pass_c(idx_arr, out_t, H_t)
    return jnp.transpose(out_t, (1, 0, 2))


# speedup vs baseline: 1.8770x; 1.0092x over previous
import functools

import jax
import jax.numpy as jnp
from jax import lax
from jax.experimental import pallas as pl
from jax.experimental.pallas import tpu as pltpu
from jax.experimental.pallas import tpu_sc as plsc

L_E = 32
N_ROWS = 729
D = 1152
EPS = 1e-8

_RC = 27
_NR = N_ROWS // _RC


def _copy_stats_body(h_ref, x_ref, o_ref, s_ref, dacc, qacc):
    r = pl.program_id(0)
    x = x_ref[...]
    o_ref[...] = x
    h = h_ref[...]

    @pl.when(r == 0)
    def _():
        dacc[...] = jnp.zeros_like(dacc)
        qacc[...] = jnp.zeros_like(qacc)

    dacc[...] += jnp.sum(x * h, axis=0)
    qacc[...] += jnp.sum(x * x, axis=0)

    @pl.when(r == _NR - 1)
    def _():
        dots = jnp.sum(dacc[...], axis=1)
        sqs = jnp.sum(qacc[...], axis=1)
        s_ref[...] = dots / (jnp.sqrt(sqs) + EPS)


def _pass_a(mem_t, h3):
    return pl.pallas_call(
        _copy_stats_body,
        grid=(_NR,),
        in_specs=[
            pl.BlockSpec((_RC, 1, D), lambda r: (r, 0, 0)),
            pl.BlockSpec((_RC, L_E, D), lambda r: (r, 0, 0)),
        ],
        out_specs=[
            pl.BlockSpec((_RC, L_E, D), lambda r: (r, 0, 0)),
            pl.BlockSpec((L_E,), lambda r: (0,)),
        ],
        out_shape=[
            jax.ShapeDtypeStruct((N_ROWS, L_E, D), jnp.float32),
            jax.ShapeDtypeStruct((L_E,), jnp.float32),
        ],
        scratch_shapes=[
            pltpu.VMEM((L_E, D), jnp.float32),
            pltpu.VMEM((L_E, D), jnp.float32),
        ],
    )(h3, mem_t)


_SC_MESH = plsc.VectorSubcoreMesh(core_axis_name="c", subcore_axis_name="s")


@functools.partial(
    pl.kernel,
    mesh=_SC_MESH,
    out_type=jax.ShapeDtypeStruct((16,), jnp.int32),
    scratch_types=[
        pltpu.VMEM((L_E,), jnp.float32),
        pltpu.VMEM((16,), jnp.int32),
    ],
)
def _sc_argmax(scores_hbm, idx_hbm, scores_v, idx_v):
    c = lax.axis_index("c")
    s = lax.axis_index("s")
    wid = s * 2 + c

    @pl.when(wid == 0)
    def _():
        pltpu.sync_copy(scores_hbm, scores_v)
        va = scores_v[pl.ds(0, 16)]
        vb = scores_v[pl.ds(16, 16)]
        svals = [va[j] for j in range(16)] + [vb[j] for j in range(16)]
        best = svals[0]
        idx = jnp.int32(0)
        for j in range(1, L_E):
            take = svals[j] > best
            best = jnp.where(take, svals[j], best)
            idx = jnp.where(take, jnp.int32(j), idx)
        idx_v[...] = jnp.broadcast_to(idx, (16,))
        pltpu.sync_copy(idx_v, idx_hbm)


def _scatter_body(idx_ref, oin_ref, h_ref, o_ref, sem):
    del oin_ref
    idx = idx_ref[0]
    cp = pltpu.make_async_copy(h_ref, o_ref.at[:, idx, :], sem)
    cp.start()
    cp.wait()


def _pass_c(idx_arr, out_t, H_t):
    return pl.pallas_call(
        _scatter_body,
        in_specs=[
            pl.BlockSpec(memory_space=pltpu.SMEM),
            pl.BlockSpec(memory_space=pl.ANY),
            pl.BlockSpec(memory_space=pltpu.VMEM),
        ],
        out_specs=pl.BlockSpec(memory_space=pl.ANY),
        out_shape=jax.ShapeDtypeStruct((N_ROWS, L_E, D), jnp.float32),
        scratch_shapes=[
            pltpu.SemaphoreType.DMA,
        ],
        input_output_aliases={1: 0},
    )(idx_arr, out_t, H_t)


def kernel(mem, H_t):
    mem_t = jnp.transpose(mem, (1, 0, 2))
    h3 = H_t[:, None, :]
    out_t, scores = _pass_a(mem_t, h3)
    idx_arr = _sc_argmax(scores)
    out_t = _pass_c(idx_arr, out_t, H_t)
    return jnp.transpose(out_t, (1, 0, 2))
